# baseline (speedup 1.0000x reference)
import jax
import jax.numpy as jnp
from jax.experimental import pallas as pl
from jax.experimental.pallas import tpu as pltpu

_ROWS = 16384
_COLS = 2048
_BLK_R = 512
_GRID = _ROWS // _BLK_R
_EPS = 1e-6


def _tc_body(mask_ref, yh_ref, y_ref, out_ref, acc_ref, cnt_ref):
    i = pl.program_id(0)

    @pl.when(i == 0)
    def _init():
        acc_ref[0, 0] = 0.0
        cnt_ref[0, 0] = 0.0

    yh = yh_ref[...]
    yy = y_ref[...]
    w = mask_ref[0]
    m = (yy != 0.0) & (w > 0.0)
    d = yh - yy
    acc_ref[0, 0] += jnp.sum(jnp.where(m, d * d, 0.0))
    cnt_ref[0, 0] += jnp.sum(jnp.where(m, 1.0, 0.0))

    @pl.when(i == _GRID - 1)
    def _fin():
        out_ref[0, 0] = jnp.sqrt(acc_ref[0, 0] / cnt_ref[0, 0] + _EPS)


def kernel(yhat, y, batch_mask):
    maskf = batch_mask.astype(jnp.float32).reshape(_GRID, _BLK_R, 1)
    out = pl.pallas_call(
        _tc_body,
        grid=(_GRID,),
        in_specs=[
            pl.BlockSpec((1, _BLK_R, 1), lambda i: (i, 0, 0)),
            pl.BlockSpec((_BLK_R, _COLS), lambda i: (i, 0)),
            pl.BlockSpec((_BLK_R, _COLS), lambda i: (i, 0)),
        ],
        out_specs=pl.BlockSpec((1, 1), lambda i: (0, 0), memory_space=pltpu.SMEM),
        out_shape=jax.ShapeDtypeStruct((1, 1), jnp.float32),
        scratch_shapes=[
            pltpu.SMEM((1, 1), jnp.float32),
            pltpu.SMEM((1, 1), jnp.float32),
        ],
    )(maskf, yhat, y)
    return out.reshape(())
